# baseline (speedup 1.0000x reference)
import functools

import jax
import jax.numpy as jnp
from jax import lax
from jax.experimental import pallas as pl
from jax.experimental.pallas import tpu as pltpu
from jax.experimental.pallas import tpu_sc as plsc

ALPHA = 4

_NC = 2
_NS = 16
_NW = _NC * _NS


def kernel(frames):
    C, T, H, W = frames.shape
    n_slow = T // ALPHA
    idx = jnp.linspace(0.0, float(T - 1), n_slow).astype(jnp.int32)

    SPLIT = 16
    row_len = H * W // SPLIT
    src_rows = C * T * SPLIT
    out_rows = C * n_slow * SPLIT
    rpw = out_rows // _NW

    c_arr = jnp.arange(C, dtype=jnp.int32)[:, None, None]
    k_arr = jnp.arange(SPLIT, dtype=jnp.int32)[None, None, :]
    row_idx = ((c_arr * T + idx[None, :, None]) * SPLIT + k_arr).reshape(-1)

    src = frames.reshape(src_rows, row_len)

    mesh = plsc.VectorSubcoreMesh(core_axis_name="c", subcore_axis_name="s")

    @functools.partial(
        pl.kernel,
        mesh=mesh,
        out_type=jax.ShapeDtypeStruct((out_rows, row_len), frames.dtype),
        scratch_types=[
            pltpu.VMEM((rpw,), jnp.int32),
            pltpu.VMEM((rpw, row_len), frames.dtype),
            pltpu.SemaphoreType.DMA,
        ],
    )
    def sc_gather(src_hbm, idx_hbm, out_hbm, idx_v, rows_v, sem):
        wid = lax.axis_index("s") * _NC + lax.axis_index("c")
        base = wid * rpw
        pltpu.sync_copy(idx_hbm.at[pl.ds(base, rpw)], idx_v)
        pltpu.async_copy(src_hbm.at[idx_v], rows_v, sem).wait()
        pltpu.sync_copy(rows_v, out_hbm.at[pl.ds(base, rpw)])

    slow = sc_gather(src, row_idx).reshape(C, n_slow, H, W)
    return (slow, frames)
